# Initial kernel scaffold; baseline (speedup 1.0000x reference)
import jax
import jax.numpy as jnp
from jax.experimental import pallas as pl


def kernel(x_paper, x_author, cites_edge_index, writes_src, writes_dst, rev_src, rev_dst, gcn1_W, gcn1_b, s1w_Wl, s1w_bl, s1w_Wr, s1r_Wl, s1r_bl, s1r_Wr, s2c_Wl, s2c_bl, s2c_Wr, s2w_Wl, s2w_bl, s2w_Wr, s2r_Wl, s2r_bl, s2r_Wr, gcn3_W, gcn3_b, s3w_Wl, s3w_bl, s3w_Wr, s3r_Wl, s3r_bl, s3r_Wr, lin_W, lin_b):
    raise NotImplementedError("write your pallas kernel here")


# speedup vs baseline: 2.1626x; 2.1626x over previous
import functools

import jax
import jax.numpy as jnp
from jax import lax
from jax.experimental import pallas as pl

NP_ = 10000
NA = 5000
DIN = 128
H = 256
OUT = 128


def _matmul_bias_kernel(x_ref, w_ref, b_ref, o_ref):
    o_ref[...] = jnp.dot(x_ref[...], w_ref[...],
                         preferred_element_type=jnp.float32) + b_ref[...]


def _final_linear(x, w, b):
    R = 1000
    grid = (x.shape[0] // R,)
    return pl.pallas_call(
        _matmul_bias_kernel,
        grid=grid,
        in_specs=[
            pl.BlockSpec((R, x.shape[1]), lambda i: (i, 0)),
            pl.BlockSpec((x.shape[1], w.shape[1]), lambda i: (0, 0)),
            pl.BlockSpec((1, w.shape[1]), lambda i: (0, 0)),
        ],
        out_specs=pl.BlockSpec((R, w.shape[1]), lambda i: (i, 0)),
        out_shape=jax.ShapeDtypeStruct((x.shape[0], w.shape[1]), jnp.float32),
    )(x, w, b.reshape(1, -1))


def _seg(table, src, dst, n):
    return jax.ops.segment_sum(table[src], dst, num_segments=n)


def _cnt(dst, n):
    return jax.ops.segment_sum(jnp.ones(dst.shape[0], jnp.float32), dst,
                               num_segments=n)


def kernel(x_paper, x_author, cites_edge_index, writes_src, writes_dst,
           rev_src, rev_dst,
           gcn1_W, gcn1_b, s1w_Wl, s1w_bl, s1w_Wr, s1r_Wl, s1r_bl, s1r_Wr,
           s2c_Wl, s2c_bl, s2c_Wr, s2w_Wl, s2w_bl, s2w_Wr,
           s2r_Wl, s2r_bl, s2r_Wr,
           gcn3_W, gcn3_b, s3w_Wl, s3w_bl, s3w_Wr, s3r_Wl, s3r_bl, s3r_Wr,
           lin_W, lin_b):
    cs, cd = cites_edge_index[0], cites_edge_index[1]
    ws, wd = writes_src, writes_dst
    rs, rd = rev_src, rev_dst
    cnt_c = _cnt(cd, NP_)
    cnt_w = _cnt(wd, NP_)
    cnt_r = _cnt(rd, NA)
    dinv = lax.rsqrt(cnt_c + 1.0)
    iw = 1.0 / jnp.maximum(cnt_w, 1.0)
    ir = 1.0 / jnp.maximum(cnt_r, 1.0)
    ic = 1.0 / jnp.maximum(cnt_c, 1.0)

    z = dinv[:, None] * x_paper
    S1c = _seg(z, cs, cd, NP_)
    S1w = _seg(x_author, ws, wd, NP_)
    S1r = _seg(x_paper, rs, rd, NA)
    p = (dinv[:, None] * (S1c + z)) @ gcn1_W + gcn1_b \
        + (iw[:, None] * S1w) @ s1w_Wl + s1w_bl + x_paper @ s1w_Wr
    a = (ir[:, None] * S1r) @ s1r_Wl + s1r_bl + x_author @ s1r_Wr
    p = jax.nn.relu(p)
    a = jax.nn.relu(a)

    S2c = _seg(p, cs, cd, NP_)
    S2w = _seg(a, ws, wd, NP_)
    S2r = _seg(p, rs, rd, NA)
    p2 = (ic[:, None] * S2c) @ s2c_Wl + s2c_bl + p @ s2c_Wr \
        + (iw[:, None] * S2w) @ s2w_Wl + s2w_bl + p @ s2w_Wr
    a2 = (ir[:, None] * S2r) @ s2r_Wl + s2r_bl + a @ s2r_Wr
    p2 = jax.nn.relu(p2)
    a2 = jax.nn.relu(a2)

    z3 = dinv[:, None] * p2
    S3c = _seg(z3, cs, cd, NP_)
    S3w = _seg(a2, ws, wd, NP_)
    p3 = (dinv[:, None] * (S3c + z3)) @ gcn3_W + gcn3_b \
        + (iw[:, None] * S3w) @ s3w_Wl + s3w_bl + p2 @ s3w_Wr
    p3 = jax.nn.relu(p3)
    return _final_linear(p3, lin_W, lin_b)


# speedup vs baseline: 4.2303x; 1.9561x over previous
import functools

import jax
import jax.numpy as jnp
from jax import lax
from jax.experimental import pallas as pl
from jax.experimental.pallas import tpu as pltpu
from jax.experimental.pallas import tpu_sc as plsc

NP_ = 10000
NA = 5000
DIN = 128
H = 256
OUT = 128

NSUB = 16
NCORE = 2
K = 128

NPP = 10112
NAP = 5120
ECP = 323584
EWP = 163840


def _seg_kernel_body(Ns, Nd_pad, E_pad, col_split,
                     tbl, src, dst, zeros, out, sidx, didx, rows, acc, sem):
    rows_per_sub = Nd_pad // NSUB
    c = lax.axis_index("c")
    s = lax.axis_index("s")
    r0 = s * rows_per_sub
    pltpu.sync_copy(zeros, acc.at[pl.ds(r0, rows_per_sub)])
    plsc.subcore_barrier()
    if col_split:
        nchunks = E_pad // (NSUB * K)
        base = s * (nchunks * K)
        off = c * Ns
    else:
        nchunks = E_pad // (NCORE * NSUB * K)
        base = (c * NSUB + s) * (nchunks * K)
        off = None

    def chunk(i, carry):
        e0 = pl.multiple_of(base + i * K, K)
        pltpu.sync_copy(src.at[pl.ds(e0, K)], sidx)
        if off is not None:
            for j in range(K // 16):
                sidx[pl.ds(j * 16, 16)] = sidx[pl.ds(j * 16, 16)] + off
        pltpu.sync_copy(dst.at[pl.ds(e0, K)], didx)
        pltpu.async_copy(tbl.at[sidx], rows, sem).wait()
        pltpu.sync_copy(rows, acc.at[didx], add=True)
        return carry

    lax.fori_loop(0, nchunks, chunk, 0)
    plsc.subcore_barrier()
    pltpu.sync_copy(acc.at[pl.ds(r0, rows_per_sub)],
                    out.at[pl.ds(c * Nd_pad + r0, rows_per_sub)])


@functools.lru_cache(maxsize=None)
def _make_seg(Ns, Nd_pad, E_pad, col_split):
    mesh = plsc.VectorSubcoreMesh(core_axis_name="c", subcore_axis_name="s")
    body = functools.partial(_seg_kernel_body, Ns, Nd_pad, E_pad, col_split)
    return pl.kernel(
        body,
        out_type=jax.ShapeDtypeStruct((NCORE * Nd_pad, 128), jnp.float32),
        mesh=mesh,
        scratch_types=[
            pltpu.VMEM((K,), jnp.int32),
            pltpu.VMEM((K,), jnp.int32),
            pltpu.VMEM((K, 128), jnp.float32),
            pltpu.VMEM_SHARED((Nd_pad, 128), jnp.float32),
            pltpu.SemaphoreType.DMA,
        ],
    )


NW = NCORE * NSUB


def _cnt_kernel_body(dst_c, dst_w, dst_r, ones, zc,
                     outc, outw, outr, didx, ones_v, acc):
    c = lax.axis_index("c")
    s = lax.axis_index("s")
    rp = NPP // NSUB
    pltpu.sync_copy(ones, ones_v)

    def count_rel(dst_ref, out_ref, e_pad, nd_pad):
        npt = e_pad // NW
        base = (c * NSUB + s) * npt
        pltpu.sync_copy(zc, acc.at[pl.ds(s * rp, rp)])
        plsc.subcore_barrier()

        def chunk(i, carry):
            e0 = pl.multiple_of(base + i * K, K)
            pltpu.sync_copy(dst_ref.at[pl.ds(e0, K)], didx)
            pltpu.sync_copy(ones_v, acc.at[didx], add=True)
            return carry

        lax.fori_loop(0, npt // K, chunk, 0)
        plsc.subcore_barrier()
        rps = nd_pad // NSUB
        pltpu.sync_copy(acc.at[pl.ds(s * rps, rps)],
                        out_ref.at[pl.ds(c * nd_pad + s * rps, rps)])
        plsc.subcore_barrier()

    count_rel(dst_c, outc, ECP, NPP)
    count_rel(dst_w, outw, EWP, NPP)
    count_rel(dst_r, outr, EWP, NAP)


@functools.lru_cache(maxsize=None)
def _make_cnt():
    mesh = plsc.VectorSubcoreMesh(core_axis_name="c", subcore_axis_name="s")
    return pl.kernel(
        _cnt_kernel_body,
        out_type=(
            jax.ShapeDtypeStruct((NCORE * NPP, 128), jnp.float32),
            jax.ShapeDtypeStruct((NCORE * NPP, 128), jnp.float32),
            jax.ShapeDtypeStruct((NCORE * NAP, 128), jnp.float32),
        ),
        mesh=mesh,
        scratch_types=[
            pltpu.VMEM((K,), jnp.int32),
            pltpu.VMEM((K, 128), jnp.float32),
            pltpu.VMEM_SHARED((NPP, 128), jnp.float32),
        ],
    )


def _split(x):
    n, d = x.shape
    hw = d // 2
    return x.reshape(n, 2, hw).transpose(1, 0, 2).reshape(2 * n, hw)


def _unsplit(o, n):
    nd_pad = o.shape[0] // 2
    return jnp.concatenate([o[:n], o[nd_pad:nd_pad + n]], axis=1)


def _seg_wide(x, src_pad, dst_pad, Nd, Nd_pad, E_pad):
    zeros = jnp.zeros((Nd_pad // NSUB, 128), jnp.float32)
    o = _make_seg(x.shape[0], Nd_pad, E_pad, True)(_split(x), src_pad,
                                                   dst_pad, zeros)
    return _unsplit(o, Nd)


def _seg_narrow(x, src_pad, dst_pad, Nd, Nd_pad, E_pad):
    zeros = jnp.zeros((Nd_pad // NSUB, 128), jnp.float32)
    o = _make_seg(x.shape[0], Nd_pad, E_pad, False)(x, src_pad, dst_pad,
                                                    zeros)
    return o[:Nd] + o[Nd_pad:Nd_pad + Nd]


def _pad_edges(src, dst, e_pad, nd):
    e = src.shape[0]
    src_p = jnp.concatenate([src, jnp.zeros((e_pad - e,), jnp.int32)])
    dst_p = jnp.concatenate([dst, jnp.full((e_pad - e,), nd, jnp.int32)])
    return src_p, dst_p


def _matmul_bias_kernel(x_ref, w_ref, b_ref, o_ref):
    o_ref[...] = jnp.dot(x_ref[...], w_ref[...],
                         preferred_element_type=jnp.float32) + b_ref[...]


def _final_linear(x, w, b):
    R = 1000
    grid = (x.shape[0] // R,)
    return pl.pallas_call(
        _matmul_bias_kernel,
        grid=grid,
        in_specs=[
            pl.BlockSpec((R, x.shape[1]), lambda i: (i, 0)),
            pl.BlockSpec((x.shape[1], w.shape[1]), lambda i: (0, 0)),
            pl.BlockSpec((1, w.shape[1]), lambda i: (0, 0)),
        ],
        out_specs=pl.BlockSpec((R, w.shape[1]), lambda i: (i, 0)),
        out_shape=jax.ShapeDtypeStruct((x.shape[0], w.shape[1]), jnp.float32),
    )(x, w, b.reshape(1, -1))


def kernel(x_paper, x_author, cites_edge_index, writes_src, writes_dst,
           rev_src, rev_dst,
           gcn1_W, gcn1_b, s1w_Wl, s1w_bl, s1w_Wr, s1r_Wl, s1r_bl, s1r_Wr,
           s2c_Wl, s2c_bl, s2c_Wr, s2w_Wl, s2w_bl, s2w_Wr,
           s2r_Wl, s2r_bl, s2r_Wr,
           gcn3_W, gcn3_b, s3w_Wl, s3w_bl, s3w_Wr, s3r_Wl, s3r_bl, s3r_Wr,
           lin_W, lin_b):
    cs, cd = cites_edge_index[0], cites_edge_index[1]
    cs, cd = _pad_edges(cs, cd, ECP, NP_)
    ws, wd = _pad_edges(writes_src, writes_dst, EWP, NP_)
    rs, rd = _pad_edges(rev_src, rev_dst, EWP, NA)

    ones = jnp.ones((K, 128), jnp.float32)
    zc = jnp.zeros((NPP // NSUB, 128), jnp.float32)
    pc, pw, pr = _make_cnt()(cd, wd, rd, ones, zc)
    cnt_c = pc[:NP_, 0] + pc[NPP:NPP + NP_, 0]
    cnt_w = pw[:NP_, 0] + pw[NPP:NPP + NP_, 0]
    cnt_r = pr[:NA, 0] + pr[NAP:NAP + NA, 0]

    dinv = lax.rsqrt(cnt_c + 1.0)
    iw = 1.0 / jnp.maximum(cnt_w, 1.0)
    ir = 1.0 / jnp.maximum(cnt_r, 1.0)
    ic = 1.0 / jnp.maximum(cnt_c, 1.0)

    z = dinv[:, None] * x_paper
    S1c = _seg_narrow(z, cs, cd, NP_, NPP, ECP)
    S1w = _seg_narrow(x_author, ws, wd, NP_, NPP, EWP)
    S1r = _seg_narrow(x_paper, rs, rd, NA, NAP, EWP)
    p = (dinv[:, None] * (S1c + z)) @ gcn1_W + gcn1_b \
        + (iw[:, None] * S1w) @ s1w_Wl + s1w_bl + x_paper @ s1w_Wr
    a = (ir[:, None] * S1r) @ s1r_Wl + s1r_bl + x_author @ s1r_Wr
    p = jax.nn.relu(p)
    a = jax.nn.relu(a)

    S2c = _seg_wide(p, cs, cd, NP_, NPP, ECP)
    S2w = _seg_wide(a, ws, wd, NP_, NPP, EWP)
    S2r = _seg_wide(p, rs, rd, NA, NAP, EWP)
    p2 = (ic[:, None] * S2c) @ s2c_Wl + s2c_bl + p @ s2c_Wr \
        + (iw[:, None] * S2w) @ s2w_Wl + s2w_bl + p @ s2w_Wr
    a2 = (ir[:, None] * S2r) @ s2r_Wl + s2r_bl + a @ s2r_Wr
    p2 = jax.nn.relu(p2)
    a2 = jax.nn.relu(a2)

    z3 = dinv[:, None] * p2
    S3c = _seg_wide(z3, cs, cd, NP_, NPP, ECP)
    S3w = _seg_wide(a2, ws, wd, NP_, NPP, EWP)
    p3 = (dinv[:, None] * (S3c + z3)) @ gcn3_W + gcn3_b \
        + (iw[:, None] * S3w) @ s3w_Wl + s3w_bl + p2 @ s3w_Wr
    p3 = jax.nn.relu(p3)
    return _final_linear(p3, lin_W, lin_b)


# speedup vs baseline: 4.8543x; 1.1475x over previous
import functools

import jax
import jax.numpy as jnp
from jax import lax
from jax.experimental import pallas as pl
from jax.experimental.pallas import tpu as pltpu
from jax.experimental.pallas import tpu_sc as plsc

NP_ = 10000
NA = 5000
DIN = 128
H = 256
OUT = 128

NSUB = 16
NCORE = 2
K = 128

NPP = 10112
NAP = 5120
ECP = 327680
EWP = 163840


def _seg_kernel_body(Ns, Nd_pad, E_pad, col_split, NB,
                     tbl, src, dst, zeros, out,
                     sblk0, sblk1, dblk0, dblk1, rows0, rows1, acc,
                     issem0, issem1, idsem0, idsem1, gsem0, gsem1):
    rows_per_sub = Nd_pad // NSUB
    c = lax.axis_index("c")
    s = lax.axis_index("s")
    r0 = s * rows_per_sub
    pltpu.sync_copy(zeros, acc.at[pl.ds(r0, rows_per_sub)])
    if col_split:
        nchunks = E_pad // (NSUB * K)
        sbase = c * (E_pad // K) + s * nchunks
        dbase = s * nchunks
    else:
        nchunks = E_pad // (NCORE * NSUB * K)
        sbase = (c * NSUB + s) * nchunks
        dbase = sbase
    nblocks = nchunks // NB
    sblks = (sblk0, sblk1)
    dblks = (dblk0, dblk1)
    issems = (issem0, issem1)
    idsems = (idsem0, idsem1)
    rows = (rows0, rows1)
    gsems = (gsem0, gsem1)

    def idx_start(g, bb):
        pltpu.async_copy(src.at[pl.ds(sbase + g * NB, NB)],
                         sblks[bb], issems[bb])
        pltpu.async_copy(dst.at[pl.ds(dbase + g * NB, NB)],
                         dblks[bb], idsems[bb])

    idx_start(0, 0)
    plsc.subcore_barrier()

    def run_block(g, bb):
        pltpu.make_async_copy(src.at[pl.ds(0, NB)],
                              sblks[bb], issems[bb]).wait()
        pltpu.make_async_copy(dst.at[pl.ds(0, NB)],
                              dblks[bb], idsems[bb]).wait()

        @pl.when(g + 1 < nblocks)
        def _():
            idx_start(g + 1, 1 - bb)

        pltpu.async_copy(tbl.at[sblks[bb].at[0]], rows0, gsem0)

        def inner(t, carry):
            for rb in range(2):
                j = 2 * t + rb

                @pl.when(j + 1 < NB)
                def _():
                    pltpu.async_copy(tbl.at[sblks[bb].at[j + 1]],
                                     rows[1 - rb], gsems[1 - rb])

                pltpu.make_async_copy(tbl.at[sblks[bb].at[j]],
                                      rows[rb], gsems[rb]).wait()
                pltpu.sync_copy(rows[rb], acc.at[dblks[bb].at[j]], add=True)
            return carry

        lax.fori_loop(0, NB // 2, inner, 0)

    def pair(h, carry):
        run_block(2 * h, 0)
        run_block(2 * h + 1, 1)
        return carry

    lax.fori_loop(0, nblocks // 2, pair, 0)
    if nblocks % 2:
        run_block(nblocks - 1, 0)
    plsc.subcore_barrier()
    pltpu.sync_copy(acc.at[pl.ds(r0, rows_per_sub)],
                    out.at[pl.ds(c * Nd_pad + r0, rows_per_sub)])


@functools.lru_cache(maxsize=None)
def _make_seg(Ns, Nd_pad, E_pad, col_split):
    mesh = plsc.VectorSubcoreMesh(core_axis_name="c", subcore_axis_name="s")
    nchunks = (E_pad // (NSUB * K) if col_split
               else E_pad // (NCORE * NSUB * K))
    NB = 16 if nchunks % 16 == 0 else 8
    body = functools.partial(_seg_kernel_body, Ns, Nd_pad, E_pad, col_split,
                             NB)
    return pl.kernel(
        body,
        out_type=jax.ShapeDtypeStruct((NCORE * Nd_pad, 128), jnp.float32),
        mesh=mesh,
        scratch_types=[
            pltpu.VMEM((NB, K), jnp.int32),
            pltpu.VMEM((NB, K), jnp.int32),
            pltpu.VMEM((NB, K), jnp.int32),
            pltpu.VMEM((NB, K), jnp.int32),
            pltpu.VMEM((K, 128), jnp.float32),
            pltpu.VMEM((K, 128), jnp.float32),
            pltpu.VMEM_SHARED((Nd_pad, 128), jnp.float32),
            pltpu.SemaphoreType.DMA,
            pltpu.SemaphoreType.DMA,
            pltpu.SemaphoreType.DMA,
            pltpu.SemaphoreType.DMA,
            pltpu.SemaphoreType.DMA,
            pltpu.SemaphoreType.DMA,
        ],
    )


NW = NCORE * NSUB


NCC = ECP // (NW * K)
NCW = EWP // (NW * K)


def _cnt_kernel_body(dst_c, dst_w, dst_r, ones, zc,
                     outc, outw, outr, didx_all, ones_v, acc):
    c = lax.axis_index("c")
    s = lax.axis_index("s")
    w = c * NSUB + s
    rp = NPP // NSUB
    pltpu.sync_copy(ones, ones_v)
    pltpu.sync_copy(dst_c.at[pl.ds(w * NCC, NCC)], didx_all.at[pl.ds(0, NCC)])
    pltpu.sync_copy(dst_w.at[pl.ds(w * NCW, NCW)],
                    didx_all.at[pl.ds(NCC, NCW)])
    pltpu.sync_copy(dst_r.at[pl.ds(w * NCW, NCW)],
                    didx_all.at[pl.ds(NCC + NCW, NCW)])

    def count_rel(off, nch, out_ref, nd_pad):
        pltpu.sync_copy(zc, acc.at[pl.ds(s * rp, rp)])
        plsc.subcore_barrier()

        def chunk(i, carry):
            pltpu.sync_copy(ones_v, acc.at[didx_all.at[off + i]], add=True)
            return carry

        lax.fori_loop(0, nch, chunk, 0)
        plsc.subcore_barrier()
        rps = nd_pad // NSUB
        pltpu.sync_copy(acc.at[pl.ds(s * rps, rps)],
                        out_ref.at[pl.ds(c * nd_pad + s * rps, rps)])
        plsc.subcore_barrier()

    count_rel(0, NCC, outc, NPP)
    count_rel(NCC, NCW, outw, NPP)
    count_rel(NCC + NCW, NCW, outr, NAP)


@functools.lru_cache(maxsize=None)
def _make_cnt():
    mesh = plsc.VectorSubcoreMesh(core_axis_name="c", subcore_axis_name="s")
    return pl.kernel(
        _cnt_kernel_body,
        out_type=(
            jax.ShapeDtypeStruct((NCORE * NPP, 128), jnp.float32),
            jax.ShapeDtypeStruct((NCORE * NPP, 128), jnp.float32),
            jax.ShapeDtypeStruct((NCORE * NAP, 128), jnp.float32),
        ),
        mesh=mesh,
        scratch_types=[
            pltpu.VMEM((NCC + 2 * NCW, K), jnp.int32),
            pltpu.VMEM((K, 128), jnp.float32),
            pltpu.VMEM_SHARED((NPP, 128), jnp.float32),
        ],
    )


def _split(x):
    n, d = x.shape
    hw = d // 2
    return x.reshape(n, 2, hw).transpose(1, 0, 2).reshape(2 * n, hw)


def _unsplit(o, n):
    nd_pad = o.shape[0] // 2
    return jnp.concatenate([o[:n], o[nd_pad:nd_pad + n]], axis=1)


def _seg_wide(x, src2w, dst2, Nd, Nd_pad, E_pad):
    zeros = jnp.zeros((Nd_pad // NSUB, 128), jnp.float32)
    o = _make_seg(x.shape[0], Nd_pad, E_pad, True)(_split(x), src2w,
                                                   dst2, zeros)
    return _unsplit(o, Nd)


def _seg_narrow(x, src2, dst2, Nd, Nd_pad, E_pad):
    zeros = jnp.zeros((Nd_pad // NSUB, 128), jnp.float32)
    o = _make_seg(x.shape[0], Nd_pad, E_pad, False)(x, src2, dst2, zeros)
    return o[:Nd] + o[Nd_pad:Nd_pad + Nd]


def _pad_edges(src, dst, e_pad, nd):
    e = src.shape[0]
    src_p = jnp.concatenate([src, jnp.zeros((e_pad - e,), jnp.int32)])
    dst_p = jnp.concatenate([dst, jnp.full((e_pad - e,), nd, jnp.int32)])
    return src_p, dst_p


def _matmul_bias_kernel(x_ref, w_ref, b_ref, o_ref):
    o_ref[...] = jnp.dot(x_ref[...], w_ref[...],
                         preferred_element_type=jnp.float32) + b_ref[...]


def _final_linear(x, w, b):
    R = 1000
    grid = (x.shape[0] // R,)
    return pl.pallas_call(
        _matmul_bias_kernel,
        grid=grid,
        in_specs=[
            pl.BlockSpec((R, x.shape[1]), lambda i: (i, 0)),
            pl.BlockSpec((x.shape[1], w.shape[1]), lambda i: (0, 0)),
            pl.BlockSpec((1, w.shape[1]), lambda i: (0, 0)),
        ],
        out_specs=pl.BlockSpec((R, w.shape[1]), lambda i: (i, 0)),
        out_shape=jax.ShapeDtypeStruct((x.shape[0], w.shape[1]), jnp.float32),
    )(x, w, b.reshape(1, -1))


def kernel(x_paper, x_author, cites_edge_index, writes_src, writes_dst,
           rev_src, rev_dst,
           gcn1_W, gcn1_b, s1w_Wl, s1w_bl, s1w_Wr, s1r_Wl, s1r_bl, s1r_Wr,
           s2c_Wl, s2c_bl, s2c_Wr, s2w_Wl, s2w_bl, s2w_Wr,
           s2r_Wl, s2r_bl, s2r_Wr,
           gcn3_W, gcn3_b, s3w_Wl, s3w_bl, s3w_Wr, s3r_Wl, s3r_bl, s3r_Wr,
           lin_W, lin_b):
    cs, cd = cites_edge_index[0], cites_edge_index[1]
    cs, cd = _pad_edges(cs, cd, ECP, NP_)
    ws, wd = _pad_edges(writes_src, writes_dst, EWP, NP_)
    rs, rd = _pad_edges(rev_src, rev_dst, EWP, NA)
    cs2, cd2 = cs.reshape(-1, K), cd.reshape(-1, K)
    ws2, wd2 = ws.reshape(-1, K), wd.reshape(-1, K)
    rs2, rd2 = rs.reshape(-1, K), rd.reshape(-1, K)
    csw = jnp.concatenate([cs, cs + NP_]).reshape(-1, K)
    wsw = jnp.concatenate([ws, ws + NA]).reshape(-1, K)
    rsw = jnp.concatenate([rs, rs + NP_]).reshape(-1, K)

    ones = jnp.ones((K, 128), jnp.float32)
    zc = jnp.zeros((NPP // NSUB, 128), jnp.float32)
    pc, pw, pr = _make_cnt()(cd2, wd2, rd2, ones, zc)
    cnt_c = pc[:NP_, 0] + pc[NPP:NPP + NP_, 0]
    cnt_w = pw[:NP_, 0] + pw[NPP:NPP + NP_, 0]
    cnt_r = pr[:NA, 0] + pr[NAP:NAP + NA, 0]

    dinv = lax.rsqrt(cnt_c + 1.0)
    iw = 1.0 / jnp.maximum(cnt_w, 1.0)
    ir = 1.0 / jnp.maximum(cnt_r, 1.0)
    ic = 1.0 / jnp.maximum(cnt_c, 1.0)

    z = dinv[:, None] * x_paper
    S1c = _seg_narrow(z, cs2, cd2, NP_, NPP, ECP)
    S1w = _seg_narrow(x_author, ws2, wd2, NP_, NPP, EWP)
    S1r = _seg_narrow(x_paper, rs2, rd2, NA, NAP, EWP)
    p = (dinv[:, None] * (S1c + z)) @ gcn1_W + gcn1_b \
        + (iw[:, None] * S1w) @ s1w_Wl + s1w_bl + x_paper @ s1w_Wr
    a = (ir[:, None] * S1r) @ s1r_Wl + s1r_bl + x_author @ s1r_Wr
    p = jax.nn.relu(p)
    a = jax.nn.relu(a)

    S2c = _seg_wide(p, csw, cd2, NP_, NPP, ECP)
    S2w = _seg_wide(a, wsw, wd2, NP_, NPP, EWP)
    S2r = _seg_wide(p, rsw, rd2, NA, NAP, EWP)
    p2 = (ic[:, None] * S2c) @ s2c_Wl + s2c_bl + p @ s2c_Wr \
        + (iw[:, None] * S2w) @ s2w_Wl + s2w_bl + p @ s2w_Wr
    a2 = (ir[:, None] * S2r) @ s2r_Wl + s2r_bl + a @ s2r_Wr
    p2 = jax.nn.relu(p2)
    a2 = jax.nn.relu(a2)

    z3 = dinv[:, None] * p2
    S3c = _seg_wide(z3, csw, cd2, NP_, NPP, ECP)
    S3w = _seg_wide(a2, wsw, wd2, NP_, NPP, EWP)
    p3 = (dinv[:, None] * (S3c + z3)) @ gcn3_W + gcn3_b \
        + (iw[:, None] * S3w) @ s3w_Wl + s3w_bl + p2 @ s3w_Wr
    p3 = jax.nn.relu(p3)
    return _final_linear(p3, lin_W, lin_b)


# speedup vs baseline: 5.1198x; 1.0547x over previous
import functools

import jax
import jax.numpy as jnp
from jax import lax
from jax.experimental import pallas as pl
from jax.experimental.pallas import tpu as pltpu
from jax.experimental.pallas import tpu_sc as plsc

NP_ = 10000
NA = 5000
DIN = 128
H = 256
OUT = 128

NSUB = 16
NCORE = 2
K = 128

NPP = 10112
NAP = 5120
ECP = 327680
EWP = 163840


def _seg_kernel_body(Ns, Nd_pad, E_pad, col_split, NB,
                     tbl, src, dst, zeros, out,
                     sblk0, sblk1, dblk0, dblk1, rows0, rows1, acc,
                     issem0, issem1, idsem0, idsem1, gsem0, gsem1):
    rows_per_sub = Nd_pad // NSUB
    c = lax.axis_index("c")
    s = lax.axis_index("s")
    r0 = s * rows_per_sub
    pltpu.sync_copy(zeros, acc.at[pl.ds(r0, rows_per_sub)])
    if col_split:
        nchunks = E_pad // (NSUB * K)
        sbase = c * (E_pad // K) + s * nchunks
        dbase = s * nchunks
    else:
        nchunks = E_pad // (NCORE * NSUB * K)
        sbase = (c * NSUB + s) * nchunks
        dbase = sbase
    nblocks = nchunks // NB
    sblks = (sblk0, sblk1)
    dblks = (dblk0, dblk1)
    issems = (issem0, issem1)
    idsems = (idsem0, idsem1)
    rows = (rows0, rows1)
    gsems = (gsem0, gsem1)

    def idx_start(g, bb):
        pltpu.async_copy(src.at[pl.ds(sbase + g * NB, NB)],
                         sblks[bb], issems[bb])
        pltpu.async_copy(dst.at[pl.ds(dbase + g * NB, NB)],
                         dblks[bb], idsems[bb])

    idx_start(0, 0)
    plsc.subcore_barrier()

    def run_block(g, bb):
        pltpu.make_async_copy(src.at[pl.ds(0, NB)],
                              sblks[bb], issems[bb]).wait()
        pltpu.make_async_copy(dst.at[pl.ds(0, NB)],
                              dblks[bb], idsems[bb]).wait()

        @pl.when(g + 1 < nblocks)
        def _():
            idx_start(g + 1, 1 - bb)

        pltpu.async_copy(tbl.at[sblks[bb].at[0]], rows0, gsem0)

        def inner(t, carry):
            for rb in range(2):
                j = 2 * t + rb

                @pl.when(j + 1 < NB)
                def _():
                    pltpu.async_copy(tbl.at[sblks[bb].at[j + 1]],
                                     rows[1 - rb], gsems[1 - rb])

                pltpu.make_async_copy(tbl.at[sblks[bb].at[j]],
                                      rows[rb], gsems[rb]).wait()
                pltpu.sync_copy(rows[rb], acc.at[dblks[bb].at[j]], add=True)
            return carry

        lax.fori_loop(0, NB // 2, inner, 0)

    def pair(h, carry):
        run_block(2 * h, 0)
        run_block(2 * h + 1, 1)
        return carry

    lax.fori_loop(0, nblocks // 2, pair, 0)
    if nblocks % 2:
        run_block(nblocks - 1, 0)
    plsc.subcore_barrier()
    pltpu.sync_copy(acc.at[pl.ds(r0, rows_per_sub)],
                    out.at[pl.ds(c * Nd_pad + r0, rows_per_sub)])


@functools.lru_cache(maxsize=None)
def _make_seg(Ns, Nd_pad, E_pad, col_split):
    mesh = plsc.VectorSubcoreMesh(core_axis_name="c", subcore_axis_name="s")
    nchunks = (E_pad // (NSUB * K) if col_split
               else E_pad // (NCORE * NSUB * K))
    NB = 16 if nchunks % 16 == 0 else 8
    body = functools.partial(_seg_kernel_body, Ns, Nd_pad, E_pad, col_split,
                             NB)
    return pl.kernel(
        body,
        out_type=jax.ShapeDtypeStruct((NCORE * Nd_pad, 128), jnp.float32),
        mesh=mesh,
        scratch_types=[
            pltpu.VMEM((NB, K), jnp.int32),
            pltpu.VMEM((NB, K), jnp.int32),
            pltpu.VMEM((NB, K), jnp.int32),
            pltpu.VMEM((NB, K), jnp.int32),
            pltpu.VMEM((K, 128), jnp.float32),
            pltpu.VMEM((K, 128), jnp.float32),
            pltpu.VMEM_SHARED((Nd_pad, 128), jnp.float32),
            pltpu.SemaphoreType.DMA,
            pltpu.SemaphoreType.DMA,
            pltpu.SemaphoreType.DMA,
            pltpu.SemaphoreType.DMA,
            pltpu.SemaphoreType.DMA,
            pltpu.SemaphoreType.DMA,
        ],
    )


NW = NCORE * NSUB


NCC = ECP // (NW * K)
NCW = EWP // (NW * K)


def _cnt_kernel_body(dst_c, dst_w, dst_r, ones, zc,
                     outc, outw, outr, didx_all, ones_v, acc):
    c = lax.axis_index("c")
    s = lax.axis_index("s")
    w = c * NSUB + s
    rp = NPP // NSUB
    pltpu.sync_copy(ones, ones_v)
    pltpu.sync_copy(dst_c.at[pl.ds(w * NCC, NCC)], didx_all.at[pl.ds(0, NCC)])
    pltpu.sync_copy(dst_w.at[pl.ds(w * NCW, NCW)],
                    didx_all.at[pl.ds(NCC, NCW)])
    pltpu.sync_copy(dst_r.at[pl.ds(w * NCW, NCW)],
                    didx_all.at[pl.ds(NCC + NCW, NCW)])

    def count_rel(off, nch, out_ref, nd_pad):
        pltpu.sync_copy(zc, acc.at[pl.ds(s * rp, rp)])
        plsc.subcore_barrier()

        def chunk(i, carry):
            pltpu.sync_copy(ones_v, acc.at[didx_all.at[off + i]], add=True)
            return carry

        lax.fori_loop(0, nch, chunk, 0)
        plsc.subcore_barrier()
        rps = nd_pad // NSUB
        pltpu.sync_copy(acc.at[pl.ds(s * rps, rps)],
                        out_ref.at[pl.ds(c * nd_pad + s * rps, rps)])
        plsc.subcore_barrier()

    count_rel(0, NCC, outc, NPP)
    count_rel(NCC, NCW, outw, NPP)
    count_rel(NCC + NCW, NCW, outr, NAP)


@functools.lru_cache(maxsize=None)
def _make_cnt():
    mesh = plsc.VectorSubcoreMesh(core_axis_name="c", subcore_axis_name="s")
    return pl.kernel(
        _cnt_kernel_body,
        out_type=(
            jax.ShapeDtypeStruct((NCORE * NPP, 128), jnp.float32),
            jax.ShapeDtypeStruct((NCORE * NPP, 128), jnp.float32),
            jax.ShapeDtypeStruct((NCORE * NAP, 128), jnp.float32),
        ),
        mesh=mesh,
        scratch_types=[
            pltpu.VMEM((NCC + 2 * NCW, K), jnp.int32),
            pltpu.VMEM((K, 128), jnp.float32),
            pltpu.VMEM_SHARED((NPP, 128), jnp.float32),
        ],
    )


def _seg_wide(tbl_split, src2w, dst2, Nd_pad, E_pad):
    zeros = jnp.zeros((Nd_pad // NSUB, 128), jnp.float32)
    Ns = tbl_split.shape[0] // 2
    return _make_seg(Ns, Nd_pad, E_pad, True)(tbl_split, src2w, dst2, zeros)


def _seg_narrow(x, src2, dst2, Nd_pad, E_pad):
    zeros = jnp.zeros((Nd_pad // NSUB, 128), jnp.float32)
    return _make_seg(x.shape[0], Nd_pad, E_pad, False)(x, src2, dst2, zeros)


def _pad_edges(src, dst, e_pad, nd):
    e = src.shape[0]
    src_p = jnp.concatenate([src, jnp.zeros((e_pad - e,), jnp.int32)])
    dst_p = jnp.concatenate([dst, jnp.full((e_pad - e,), nd, jnp.int32)])
    return src_p, dst_p


RP = 2000
RA = 1000


def _sblk(nd_pad):
    return pl.BlockSpec((2, RP, 128), lambda i: (0, i, 0))


def _t0_body(x_ref, cc_ref, z_ref):
    dinv = lax.rsqrt(cc_ref[...] + 1.0)
    z_ref[...] = dinv * x_ref[...]


def _t0(x, cc):
    return pl.pallas_call(
        _t0_body,
        grid=(NP_ // RP,),
        in_specs=[
            pl.BlockSpec((RP, 128), lambda i: (i, 0)),
            pl.BlockSpec((RP, 1), lambda i: (i, 0)),
        ],
        out_specs=pl.BlockSpec((RP, 128), lambda i: (i, 0)),
        out_shape=jax.ShapeDtypeStruct((NP_, 128), jnp.float32),
    )(x, cc)


def _dot(a, b):
    return jnp.dot(a, b, preferred_element_type=jnp.float32)


def _t1p_body(S1c, z, S1w, x, cc, cw, W1, Wl, Wr, b, out):
    dinv = lax.rsqrt(cc[...] + 1.0)
    iw = 1.0 / jnp.maximum(cw[...], 1.0)
    h = _dot(dinv * (S1c[0] + S1c[1] + z[...]), W1[...])
    h += _dot(iw * (S1w[0] + S1w[1]), Wl[...])
    h += _dot(x[...], Wr[...]) + b[...]
    res = jax.nn.relu(h)
    out[0] = res[:, :128]
    out[1] = res[:, 128:]


def _t1p(S1c, z, S1w, x, cc, cw, W1, Wl, Wr, b):
    w2 = pl.BlockSpec((128, 256), lambda i: (0, 0))
    return pl.pallas_call(
        _t1p_body,
        grid=(NP_ // RP,),
        in_specs=[
            pl.BlockSpec((2, RP, 128), lambda i: (0, i, 0)),
            pl.BlockSpec((RP, 128), lambda i: (i, 0)),
            pl.BlockSpec((2, RP, 128), lambda i: (0, i, 0)),
            pl.BlockSpec((RP, 128), lambda i: (i, 0)),
            pl.BlockSpec((RP, 1), lambda i: (i, 0)),
            pl.BlockSpec((RP, 1), lambda i: (i, 0)),
            w2, w2, w2,
            pl.BlockSpec((1, 256), lambda i: (0, 0)),
        ],
        out_specs=pl.BlockSpec((2, RP, 128), lambda i: (0, i, 0)),
        out_shape=jax.ShapeDtypeStruct((2, NP_, 128), jnp.float32),
    )(S1c.reshape(2, NPP, 128), z, S1w.reshape(2, NPP, 128), x, cc, cw,
      W1, Wl, Wr, b)


def _t1a_body(S1r, xa, cr, Wl, Wr, b, out):
    ir = 1.0 / jnp.maximum(cr[...], 1.0)
    h = _dot(ir * (S1r[0] + S1r[1]), Wl[...])
    h += _dot(xa[...], Wr[...]) + b[...]
    res = jax.nn.relu(h)
    out[0] = res[:, :128]
    out[1] = res[:, 128:]


def _t1a(S1r, xa, cr, Wl, Wr, b):
    w2 = pl.BlockSpec((128, 256), lambda i: (0, 0))
    return pl.pallas_call(
        _t1a_body,
        grid=(NA // RA,),
        in_specs=[
            pl.BlockSpec((2, RA, 128), lambda i: (0, i, 0)),
            pl.BlockSpec((RA, 128), lambda i: (i, 0)),
            pl.BlockSpec((RA, 1), lambda i: (i, 0)),
            w2, w2,
            pl.BlockSpec((1, 256), lambda i: (0, 0)),
        ],
        out_specs=pl.BlockSpec((2, RA, 128), lambda i: (0, i, 0)),
        out_shape=jax.ShapeDtypeStruct((2, NA, 128), jnp.float32),
    )(S1r.reshape(2, NAP, 128), xa, cr, Wl, Wr, b)


def _t2p_body(S2c, S2w, p, cc, cw, Wlc, Wlw, Wr, b, out_p2, out_z3):
    ic = 1.0 / jnp.maximum(cc[...], 1.0)
    iw = 1.0 / jnp.maximum(cw[...], 1.0)
    dinv = lax.rsqrt(cc[...] + 1.0)
    s2c = jnp.concatenate([S2c[0], S2c[1]], axis=1)
    s2w = jnp.concatenate([S2w[0], S2w[1]], axis=1)
    pf = jnp.concatenate([p[0], p[1]], axis=1)
    h = _dot(ic * s2c, Wlc[...]) + _dot(iw * s2w, Wlw[...])
    h += _dot(pf, Wr[...]) + b[...]
    p2 = jax.nn.relu(h)
    z3 = dinv * p2
    out_p2[0] = p2[:, :128]
    out_p2[1] = p2[:, 128:]
    out_z3[0] = z3[:, :128]
    out_z3[1] = z3[:, 128:]


def _t2p(S2c, S2w, p, cc, cw, Wlc, Wlw, Wr, b):
    w2 = pl.BlockSpec((256, 256), lambda i: (0, 0))
    blk = pl.BlockSpec((2, RP, 128), lambda i: (0, i, 0))
    return pl.pallas_call(
        _t2p_body,
        grid=(NP_ // RP,),
        in_specs=[
            blk, blk, blk,
            pl.BlockSpec((RP, 1), lambda i: (i, 0)),
            pl.BlockSpec((RP, 1), lambda i: (i, 0)),
            w2, w2, w2,
            pl.BlockSpec((1, 256), lambda i: (0, 0)),
        ],
        out_specs=(blk, blk),
        out_shape=(jax.ShapeDtypeStruct((2, NP_, 128), jnp.float32),
                   jax.ShapeDtypeStruct((2, NP_, 128), jnp.float32)),
    )(S2c.reshape(2, NPP, 128), S2w.reshape(2, NPP, 128), p, cc, cw,
      Wlc, Wlw, Wr, b)


def _t2a_body(S2r, a, cr, Wl, Wr, b, out):
    ir = 1.0 / jnp.maximum(cr[...], 1.0)
    s2r = jnp.concatenate([S2r[0], S2r[1]], axis=1)
    af = jnp.concatenate([a[0], a[1]], axis=1)
    h = _dot(ir * s2r, Wl[...]) + _dot(af, Wr[...]) + b[...]
    res = jax.nn.relu(h)
    out[0] = res[:, :128]
    out[1] = res[:, 128:]


def _t2a(S2r, a, cr, Wl, Wr, b):
    w2 = pl.BlockSpec((256, 256), lambda i: (0, 0))
    blk = pl.BlockSpec((2, RA, 128), lambda i: (0, i, 0))
    return pl.pallas_call(
        _t2a_body,
        grid=(NA // RA,),
        in_specs=[
            blk, blk,
            pl.BlockSpec((RA, 1), lambda i: (i, 0)),
            w2, w2,
            pl.BlockSpec((1, 256), lambda i: (0, 0)),
        ],
        out_specs=blk,
        out_shape=jax.ShapeDtypeStruct((2, NA, 128), jnp.float32),
    )(S2r.reshape(2, NAP, 128), a, cr, Wl, Wr, b)


def _t3_body(S3c, z3, S3w, p2, cc, cw, W3, Wl, Wr, bias, lW, lb, out):
    dinv = lax.rsqrt(cc[...] + 1.0)
    iw = 1.0 / jnp.maximum(cw[...], 1.0)
    s3c = jnp.concatenate([S3c[0], S3c[1]], axis=1)
    z3f = jnp.concatenate([z3[0], z3[1]], axis=1)
    s3w = jnp.concatenate([S3w[0], S3w[1]], axis=1)
    p2f = jnp.concatenate([p2[0], p2[1]], axis=1)
    h = _dot(dinv * (s3c + z3f), W3[...])
    h += _dot(iw * s3w, Wl[...])
    h += _dot(p2f, Wr[...]) + bias[...]
    p3 = jax.nn.relu(h)
    out[...] = _dot(p3, lW[...]) + lb[...]


def _t3(S3c, z3, S3w, p2, cc, cw, W3, Wl, Wr, bias, lW, lb):
    w2 = pl.BlockSpec((256, 256), lambda i: (0, 0))
    blk = pl.BlockSpec((2, RP, 128), lambda i: (0, i, 0))
    return pl.pallas_call(
        _t3_body,
        grid=(NP_ // RP,),
        in_specs=[
            blk, blk, blk, blk,
            pl.BlockSpec((RP, 1), lambda i: (i, 0)),
            pl.BlockSpec((RP, 1), lambda i: (i, 0)),
            w2, w2, w2,
            pl.BlockSpec((1, 256), lambda i: (0, 0)),
            pl.BlockSpec((256, OUT), lambda i: (0, 0)),
            pl.BlockSpec((1, OUT), lambda i: (0, 0)),
        ],
        out_specs=pl.BlockSpec((RP, OUT), lambda i: (i, 0)),
        out_shape=jax.ShapeDtypeStruct((NP_, OUT), jnp.float32),
    )(S3c.reshape(2, NPP, 128), z3, S3w.reshape(2, NPP, 128), p2, cc, cw,
      W3, Wl, Wr, bias, lW, lb)


def kernel(x_paper, x_author, cites_edge_index, writes_src, writes_dst,
           rev_src, rev_dst,
           gcn1_W, gcn1_b, s1w_Wl, s1w_bl, s1w_Wr, s1r_Wl, s1r_bl, s1r_Wr,
           s2c_Wl, s2c_bl, s2c_Wr, s2w_Wl, s2w_bl, s2w_Wr,
           s2r_Wl, s2r_bl, s2r_Wr,
           gcn3_W, gcn3_b, s3w_Wl, s3w_bl, s3w_Wr, s3r_Wl, s3r_bl, s3r_Wr,
           lin_W, lin_b):
    cs, cd = cites_edge_index[0], cites_edge_index[1]
    cs, cd = _pad_edges(cs, cd, ECP, NP_)
    ws, wd = _pad_edges(writes_src, writes_dst, EWP, NP_)
    rs, rd = _pad_edges(rev_src, rev_dst, EWP, NA)
    cs2, cd2 = cs.reshape(-1, K), cd.reshape(-1, K)
    ws2, wd2 = ws.reshape(-1, K), wd.reshape(-1, K)
    rs2, rd2 = rs.reshape(-1, K), rd.reshape(-1, K)
    csw = jnp.concatenate([cs, cs + NP_]).reshape(-1, K)
    wsw = jnp.concatenate([ws, ws + NA]).reshape(-1, K)
    rsw = jnp.concatenate([rs, rs + NP_]).reshape(-1, K)

    ones = jnp.ones((K, 128), jnp.float32)
    zc = jnp.zeros((NPP // NSUB, 128), jnp.float32)
    pc, pw, pr = _make_cnt()(cd2, wd2, rd2, ones, zc)
    cc = pc[:NP_, 0:1] + pc[NPP:NPP + NP_, 0:1]
    cw = pw[:NP_, 0:1] + pw[NPP:NPP + NP_, 0:1]
    cr = pr[:NA, 0:1] + pr[NAP:NAP + NA, 0:1]

    z = _t0(x_paper, cc)
    S1c = _seg_narrow(z, cs2, cd2, NPP, ECP)
    S1w = _seg_narrow(x_author, ws2, wd2, NPP, EWP)
    S1r = _seg_narrow(x_paper, rs2, rd2, NAP, EWP)
    p = _t1p(S1c, z, S1w, x_paper, cc, cw,
             gcn1_W, s1w_Wl, s1w_Wr, (gcn1_b + s1w_bl).reshape(1, -1))
    a = _t1a(S1r, x_author, cr, s1r_Wl, s1r_Wr, s1r_bl.reshape(1, -1))

    S2c = _seg_wide(p.reshape(2 * NP_, 128), csw, cd2, NPP, ECP)
    S2w = _seg_wide(a.reshape(2 * NA, 128), wsw, wd2, NPP, EWP)
    S2r = _seg_wide(p.reshape(2 * NP_, 128), rsw, rd2, NAP, EWP)
    p2, z3 = _t2p(S2c, S2w, p, cc, cw, s2c_Wl, s2w_Wl, s2c_Wr + s2w_Wr,
                  (s2c_bl + s2w_bl).reshape(1, -1))
    a2 = _t2a(S2r, a, cr, s2r_Wl, s2r_Wr, s2r_bl.reshape(1, -1))

    S3c = _seg_wide(z3.reshape(2 * NP_, 128), csw, cd2, NPP, ECP)
    S3w = _seg_wide(a2.reshape(2 * NA, 128), wsw, wd2, NPP, EWP)
    return _t3(S3c, z3, S3w, p2, cc, cw, gcn3_W, s3w_Wl, s3w_Wr,
               (gcn3_b + s3w_bl).reshape(1, -1), lin_W,
               lin_b.reshape(1, -1))


# speedup vs baseline: 5.1259x; 1.0012x over previous
import functools

import jax
import jax.numpy as jnp
from jax import lax
from jax.experimental import pallas as pl
from jax.experimental.pallas import tpu as pltpu
from jax.experimental.pallas import tpu_sc as plsc

NP_ = 10000
NA = 5000
DIN = 128
H = 256
OUT = 128

NSUB = 16
NCORE = 2
K = 128

NPP = 10112
NAP = 5120
ECP = 327680
EWP = 163840


def _seg_phase(col_split, Ns, Nd_pad, E_pad, c, s,
               tbl, src, dst, zeros, out,
               sblks, dblks, rows, acc, issems, idsems, gsems):
    rows_per_sub = Nd_pad // NSUB
    r0 = s * rows_per_sub
    pltpu.sync_copy(zeros.at[pl.ds(0, rows_per_sub)],
                    acc.at[pl.ds(r0, rows_per_sub)])
    if col_split:
        nchunks = E_pad // (NSUB * K)
        sbase = c * (E_pad // K) + s * nchunks
        dbase = s * nchunks
    else:
        nchunks = E_pad // (NCORE * NSUB * K)
        sbase = (c * NSUB + s) * nchunks
        dbase = sbase
    NB = 16 if nchunks % 16 == 0 else 8
    nblocks = nchunks // NB

    def idx_start(g, bb):
        pltpu.async_copy(src.at[pl.ds(sbase + g * NB, NB)],
                         sblks[bb].at[pl.ds(0, NB)], issems[bb])
        pltpu.async_copy(dst.at[pl.ds(dbase + g * NB, NB)],
                         dblks[bb].at[pl.ds(0, NB)], idsems[bb])

    idx_start(0, 0)
    plsc.subcore_barrier()

    def run_block(g, bb):
        pltpu.make_async_copy(src.at[pl.ds(0, NB)],
                              sblks[bb].at[pl.ds(0, NB)], issems[bb]).wait()
        pltpu.make_async_copy(dst.at[pl.ds(0, NB)],
                              dblks[bb].at[pl.ds(0, NB)], idsems[bb]).wait()

        @pl.when(g + 1 < nblocks)
        def _():
            idx_start(g + 1, 1 - bb)

        pltpu.async_copy(tbl.at[sblks[bb].at[0]], rows[0], gsems[0])

        def inner(t, carry):
            for rb in range(2):
                j = 2 * t + rb

                @pl.when(j + 1 < NB)
                def _():
                    pltpu.async_copy(tbl.at[sblks[bb].at[j + 1]],
                                     rows[1 - rb], gsems[1 - rb])

                pltpu.make_async_copy(tbl.at[sblks[bb].at[j]],
                                      rows[rb], gsems[rb]).wait()
                pltpu.sync_copy(rows[rb], acc.at[dblks[bb].at[j]], add=True)
            return carry

        lax.fori_loop(0, NB // 2, inner, 0)

    def pair(h, carry):
        run_block(2 * h, 0)
        run_block(2 * h + 1, 1)
        return carry

    lax.fori_loop(0, nblocks // 2, pair, 0)
    if nblocks % 2:
        run_block(nblocks - 1, 0)
    plsc.subcore_barrier()
    pltpu.sync_copy(acc.at[pl.ds(r0, rows_per_sub)],
                    out.at[pl.ds(c * Nd_pad + r0, rows_per_sub)])
    plsc.subcore_barrier()


def _seg_multi_body(cfgs, *refs):
    nph = len(cfgs)
    tbls = refs[0:nph]
    srcs = refs[nph:2 * nph]
    dsts = refs[2 * nph:3 * nph]
    zeros = refs[3 * nph]
    outs = refs[3 * nph + 1:4 * nph + 1]
    (sblk0, sblk1, dblk0, dblk1, rows0, rows1, acc,
     issem0, issem1, idsem0, idsem1, gsem0, gsem1) = refs[4 * nph + 1:]
    c = lax.axis_index("c")
    s = lax.axis_index("s")
    for ph, (col_split, Ns, Nd_pad, E_pad) in enumerate(cfgs):
        _seg_phase(col_split, Ns, Nd_pad, E_pad, c, s,
                   tbls[ph], srcs[ph], dsts[ph], zeros, outs[ph],
                   (sblk0, sblk1), (dblk0, dblk1), (rows0, rows1), acc,
                   (issem0, issem1), (idsem0, idsem1), (gsem0, gsem1))


@functools.lru_cache(maxsize=None)
def _make_seg_multi(cfgs):
    mesh = plsc.VectorSubcoreMesh(core_axis_name="c", subcore_axis_name="s")
    body = functools.partial(_seg_multi_body, cfgs)
    return pl.kernel(
        body,
        out_type=tuple(
            jax.ShapeDtypeStruct((NCORE * Nd_pad, 128), jnp.float32)
            for (_, _, Nd_pad, _) in cfgs),
        mesh=mesh,
        scratch_types=[
            pltpu.VMEM((16, K), jnp.int32),
            pltpu.VMEM((16, K), jnp.int32),
            pltpu.VMEM((16, K), jnp.int32),
            pltpu.VMEM((16, K), jnp.int32),
            pltpu.VMEM((K, 128), jnp.float32),
            pltpu.VMEM((K, 128), jnp.float32),
            pltpu.VMEM_SHARED((NPP, 128), jnp.float32),
            pltpu.SemaphoreType.DMA,
            pltpu.SemaphoreType.DMA,
            pltpu.SemaphoreType.DMA,
            pltpu.SemaphoreType.DMA,
            pltpu.SemaphoreType.DMA,
            pltpu.SemaphoreType.DMA,
        ],
    )


NW = NCORE * NSUB


NCC = ECP // (NW * K)
NCW = EWP // (NW * K)


def _cnt_kernel_body(dst_c, dst_w, dst_r, ones, zc,
                     outc, outw, outr, didx_all, ones_v, acc):
    c = lax.axis_index("c")
    s = lax.axis_index("s")
    w = c * NSUB + s
    rp = NPP // NSUB
    pltpu.sync_copy(ones, ones_v)
    pltpu.sync_copy(dst_c.at[pl.ds(w * NCC, NCC)], didx_all.at[pl.ds(0, NCC)])
    pltpu.sync_copy(dst_w.at[pl.ds(w * NCW, NCW)],
                    didx_all.at[pl.ds(NCC, NCW)])
    pltpu.sync_copy(dst_r.at[pl.ds(w * NCW, NCW)],
                    didx_all.at[pl.ds(NCC + NCW, NCW)])

    def count_rel(off, nch, out_ref, nd_pad):
        pltpu.sync_copy(zc, acc.at[pl.ds(s * rp, rp)])
        plsc.subcore_barrier()

        def chunk(i, carry):
            pltpu.sync_copy(ones_v, acc.at[didx_all.at[off + i]], add=True)
            return carry

        lax.fori_loop(0, nch, chunk, 0)
        plsc.subcore_barrier()
        rps = nd_pad // NSUB
        pltpu.sync_copy(acc.at[pl.ds(s * rps, rps)],
                        out_ref.at[pl.ds(c * nd_pad + s * rps, rps)])
        plsc.subcore_barrier()

    count_rel(0, NCC, outc, NPP)
    count_rel(NCC, NCW, outw, NPP)
    count_rel(NCC + NCW, NCW, outr, NAP)


@functools.lru_cache(maxsize=None)
def _make_cnt():
    mesh = plsc.VectorSubcoreMesh(core_axis_name="c", subcore_axis_name="s")
    return pl.kernel(
        _cnt_kernel_body,
        out_type=(
            jax.ShapeDtypeStruct((NCORE * NPP, 128), jnp.float32),
            jax.ShapeDtypeStruct((NCORE * NPP, 128), jnp.float32),
            jax.ShapeDtypeStruct((NCORE * NAP, 128), jnp.float32),
        ),
        mesh=mesh,
        scratch_types=[
            pltpu.VMEM((NCC + 2 * NCW, K), jnp.int32),
            pltpu.VMEM((K, 128), jnp.float32),
            pltpu.VMEM_SHARED((NPP, 128), jnp.float32),
        ],
    )


def _pad_edges(src, dst, e_pad, nd):
    e = src.shape[0]
    src_p = jnp.concatenate([src, jnp.zeros((e_pad - e,), jnp.int32)])
    dst_p = jnp.concatenate([dst, jnp.full((e_pad - e,), nd, jnp.int32)])
    return src_p, dst_p


RP = 2000
RA = 1000


def _sblk(nd_pad):
    return pl.BlockSpec((2, RP, 128), lambda i: (0, i, 0))


def _t0_body(x_ref, cc_ref, z_ref):
    dinv = lax.rsqrt(cc_ref[...] + 1.0)
    z_ref[...] = dinv * x_ref[...]


def _t0(x, cc):
    return pl.pallas_call(
        _t0_body,
        grid=(NP_ // RP,),
        in_specs=[
            pl.BlockSpec((RP, 128), lambda i: (i, 0)),
            pl.BlockSpec((RP, 1), lambda i: (i, 0)),
        ],
        out_specs=pl.BlockSpec((RP, 128), lambda i: (i, 0)),
        out_shape=jax.ShapeDtypeStruct((NP_, 128), jnp.float32),
    )(x, cc)


def _dot(a, b):
    return jnp.dot(a, b, preferred_element_type=jnp.float32)


def _t1p_body(S1c, z, S1w, x, cc, cw, W1, Wl, Wr, b, out):
    dinv = lax.rsqrt(cc[...] + 1.0)
    iw = 1.0 / jnp.maximum(cw[...], 1.0)
    h = _dot(dinv * (S1c[0] + S1c[1] + z[...]), W1[...])
    h += _dot(iw * (S1w[0] + S1w[1]), Wl[...])
    h += _dot(x[...], Wr[...]) + b[...]
    res = jax.nn.relu(h)
    out[0] = res[:, :128]
    out[1] = res[:, 128:]


def _t1p(S1c, z, S1w, x, cc, cw, W1, Wl, Wr, b):
    w2 = pl.BlockSpec((128, 256), lambda i: (0, 0))
    return pl.pallas_call(
        _t1p_body,
        grid=(NP_ // RP,),
        in_specs=[
            pl.BlockSpec((2, RP, 128), lambda i: (0, i, 0)),
            pl.BlockSpec((RP, 128), lambda i: (i, 0)),
            pl.BlockSpec((2, RP, 128), lambda i: (0, i, 0)),
            pl.BlockSpec((RP, 128), lambda i: (i, 0)),
            pl.BlockSpec((RP, 1), lambda i: (i, 0)),
            pl.BlockSpec((RP, 1), lambda i: (i, 0)),
            w2, w2, w2,
            pl.BlockSpec((1, 256), lambda i: (0, 0)),
        ],
        out_specs=pl.BlockSpec((2, RP, 128), lambda i: (0, i, 0)),
        out_shape=jax.ShapeDtypeStruct((2, NP_, 128), jnp.float32),
    )(S1c.reshape(2, NPP, 128), z, S1w.reshape(2, NPP, 128), x, cc, cw,
      W1, Wl, Wr, b)


def _t1a_body(S1r, xa, cr, Wl, Wr, b, out):
    ir = 1.0 / jnp.maximum(cr[...], 1.0)
    h = _dot(ir * (S1r[0] + S1r[1]), Wl[...])
    h += _dot(xa[...], Wr[...]) + b[...]
    res = jax.nn.relu(h)
    out[0] = res[:, :128]
    out[1] = res[:, 128:]


def _t1a(S1r, xa, cr, Wl, Wr, b):
    w2 = pl.BlockSpec((128, 256), lambda i: (0, 0))
    return pl.pallas_call(
        _t1a_body,
        grid=(NA // RA,),
        in_specs=[
            pl.BlockSpec((2, RA, 128), lambda i: (0, i, 0)),
            pl.BlockSpec((RA, 128), lambda i: (i, 0)),
            pl.BlockSpec((RA, 1), lambda i: (i, 0)),
            w2, w2,
            pl.BlockSpec((1, 256), lambda i: (0, 0)),
        ],
        out_specs=pl.BlockSpec((2, RA, 128), lambda i: (0, i, 0)),
        out_shape=jax.ShapeDtypeStruct((2, NA, 128), jnp.float32),
    )(S1r.reshape(2, NAP, 128), xa, cr, Wl, Wr, b)


def _t2p_body(S2c, S2w, p, cc, cw, Wlc, Wlw, Wr, b, out_p2, out_z3):
    ic = 1.0 / jnp.maximum(cc[...], 1.0)
    iw = 1.0 / jnp.maximum(cw[...], 1.0)
    dinv = lax.rsqrt(cc[...] + 1.0)
    s2c = jnp.concatenate([S2c[0], S2c[1]], axis=1)
    s2w = jnp.concatenate([S2w[0], S2w[1]], axis=1)
    pf = jnp.concatenate([p[0], p[1]], axis=1)
    h = _dot(ic * s2c, Wlc[...]) + _dot(iw * s2w, Wlw[...])
    h += _dot(pf, Wr[...]) + b[...]
    p2 = jax.nn.relu(h)
    z3 = dinv * p2
    out_p2[0] = p2[:, :128]
    out_p2[1] = p2[:, 128:]
    out_z3[0] = z3[:, :128]
    out_z3[1] = z3[:, 128:]


def _t2p(S2c, S2w, p, cc, cw, Wlc, Wlw, Wr, b):
    w2 = pl.BlockSpec((256, 256), lambda i: (0, 0))
    blk = pl.BlockSpec((2, RP, 128), lambda i: (0, i, 0))
    return pl.pallas_call(
        _t2p_body,
        grid=(NP_ // RP,),
        in_specs=[
            blk, blk, blk,
            pl.BlockSpec((RP, 1), lambda i: (i, 0)),
            pl.BlockSpec((RP, 1), lambda i: (i, 0)),
            w2, w2, w2,
            pl.BlockSpec((1, 256), lambda i: (0, 0)),
        ],
        out_specs=(blk, blk),
        out_shape=(jax.ShapeDtypeStruct((2, NP_, 128), jnp.float32),
                   jax.ShapeDtypeStruct((2, NP_, 128), jnp.float32)),
    )(S2c.reshape(2, NPP, 128), S2w.reshape(2, NPP, 128), p, cc, cw,
      Wlc, Wlw, Wr, b)


def _t2a_body(S2r, a, cr, Wl, Wr, b, out):
    ir = 1.0 / jnp.maximum(cr[...], 1.0)
    s2r = jnp.concatenate([S2r[0], S2r[1]], axis=1)
    af = jnp.concatenate([a[0], a[1]], axis=1)
    h = _dot(ir * s2r, Wl[...]) + _dot(af, Wr[...]) + b[...]
    res = jax.nn.relu(h)
    out[0] = res[:, :128]
    out[1] = res[:, 128:]


def _t2a(S2r, a, cr, Wl, Wr, b):
    w2 = pl.BlockSpec((256, 256), lambda i: (0, 0))
    blk = pl.BlockSpec((2, RA, 128), lambda i: (0, i, 0))
    return pl.pallas_call(
        _t2a_body,
        grid=(NA // RA,),
        in_specs=[
            blk, blk,
            pl.BlockSpec((RA, 1), lambda i: (i, 0)),
            w2, w2,
            pl.BlockSpec((1, 256), lambda i: (0, 0)),
        ],
        out_specs=blk,
        out_shape=jax.ShapeDtypeStruct((2, NA, 128), jnp.float32),
    )(S2r.reshape(2, NAP, 128), a, cr, Wl, Wr, b)


def _t3_body(S3c, z3, S3w, p2, cc, cw, W3, Wl, Wr, bias, lW, lb, out):
    dinv = lax.rsqrt(cc[...] + 1.0)
    iw = 1.0 / jnp.maximum(cw[...], 1.0)
    s3c = jnp.concatenate([S3c[0], S3c[1]], axis=1)
    z3f = jnp.concatenate([z3[0], z3[1]], axis=1)
    s3w = jnp.concatenate([S3w[0], S3w[1]], axis=1)
    p2f = jnp.concatenate([p2[0], p2[1]], axis=1)
    h = _dot(dinv * (s3c + z3f), W3[...])
    h += _dot(iw * s3w, Wl[...])
    h += _dot(p2f, Wr[...]) + bias[...]
    p3 = jax.nn.relu(h)
    out[...] = _dot(p3, lW[...]) + lb[...]


def _t3(S3c, z3, S3w, p2, cc, cw, W3, Wl, Wr, bias, lW, lb):
    w2 = pl.BlockSpec((256, 256), lambda i: (0, 0))
    blk = pl.BlockSpec((2, RP, 128), lambda i: (0, i, 0))
    return pl.pallas_call(
        _t3_body,
        grid=(NP_ // RP,),
        in_specs=[
            blk, blk, blk, blk,
            pl.BlockSpec((RP, 1), lambda i: (i, 0)),
            pl.BlockSpec((RP, 1), lambda i: (i, 0)),
            w2, w2, w2,
            pl.BlockSpec((1, 256), lambda i: (0, 0)),
            pl.BlockSpec((256, OUT), lambda i: (0, 0)),
            pl.BlockSpec((1, OUT), lambda i: (0, 0)),
        ],
        out_specs=pl.BlockSpec((RP, OUT), lambda i: (i, 0)),
        out_shape=jax.ShapeDtypeStruct((NP_, OUT), jnp.float32),
    )(S3c.reshape(2, NPP, 128), z3, S3w.reshape(2, NPP, 128), p2, cc, cw,
      W3, Wl, Wr, bias, lW, lb)


def kernel(x_paper, x_author, cites_edge_index, writes_src, writes_dst,
           rev_src, rev_dst,
           gcn1_W, gcn1_b, s1w_Wl, s1w_bl, s1w_Wr, s1r_Wl, s1r_bl, s1r_Wr,
           s2c_Wl, s2c_bl, s2c_Wr, s2w_Wl, s2w_bl, s2w_Wr,
           s2r_Wl, s2r_bl, s2r_Wr,
           gcn3_W, gcn3_b, s3w_Wl, s3w_bl, s3w_Wr, s3r_Wl, s3r_bl, s3r_Wr,
           lin_W, lin_b):
    cs, cd = cites_edge_index[0], cites_edge_index[1]
    cs, cd = _pad_edges(cs, cd, ECP, NP_)
    ws, wd = _pad_edges(writes_src, writes_dst, EWP, NP_)
    rs, rd = _pad_edges(rev_src, rev_dst, EWP, NA)
    cs2, cd2 = cs.reshape(-1, K), cd.reshape(-1, K)
    ws2, wd2 = ws.reshape(-1, K), wd.reshape(-1, K)
    rs2, rd2 = rs.reshape(-1, K), rd.reshape(-1, K)
    csw = jnp.concatenate([cs, cs + NP_]).reshape(-1, K)
    wsw = jnp.concatenate([ws, ws + NA]).reshape(-1, K)
    rsw = jnp.concatenate([rs, rs + NP_]).reshape(-1, K)

    ones = jnp.ones((K, 128), jnp.float32)
    zc = jnp.zeros((NPP // NSUB, 128), jnp.float32)
    pc, pw, pr = _make_cnt()(cd2, wd2, rd2, ones, zc)
    cc = pc[:NP_, 0:1] + pc[NPP:NPP + NP_, 0:1]
    cw = pw[:NP_, 0:1] + pw[NPP:NPP + NP_, 0:1]
    cr = pr[:NA, 0:1] + pr[NAP:NAP + NA, 0:1]

    zeros = jnp.zeros((NPP // NSUB, 128), jnp.float32)

    z = _t0(x_paper, cc)
    S1c, S1w, S1r = _make_seg_multi((
        (False, NP_, NPP, ECP),
        (False, NA, NPP, EWP),
        (False, NP_, NAP, EWP),
    ))(z, x_author, x_paper, cs2, ws2, rs2, cd2, wd2, rd2, zeros)
    p = _t1p(S1c, z, S1w, x_paper, cc, cw,
             gcn1_W, s1w_Wl, s1w_Wr, (gcn1_b + s1w_bl).reshape(1, -1))
    a = _t1a(S1r, x_author, cr, s1r_Wl, s1r_Wr, s1r_bl.reshape(1, -1))

    pt = p.reshape(2 * NP_, 128)
    S2c, S2w, S2r = _make_seg_multi((
        (True, NP_, NPP, ECP),
        (True, NA, NPP, EWP),
        (True, NP_, NAP, EWP),
    ))(pt, a.reshape(2 * NA, 128), pt, csw, wsw, rsw, cd2, wd2, rd2, zeros)
    p2, z3 = _t2p(S2c, S2w, p, cc, cw, s2c_Wl, s2w_Wl, s2c_Wr + s2w_Wr,
                  (s2c_bl + s2w_bl).reshape(1, -1))
    a2 = _t2a(S2r, a, cr, s2r_Wl, s2r_Wr, s2r_bl.reshape(1, -1))

    S3c, S3w = _make_seg_multi((
        (True, NP_, NPP, ECP),
        (True, NA, NPP, EWP),
    ))(z3.reshape(2 * NP_, 128), a2.reshape(2 * NA, 128),
       csw, wsw, cd2, wd2, zeros)
    return _t3(S3c, z3, S3w, p2, cc, cw, gcn3_W, s3w_Wl, s3w_Wr,
               (gcn3_b + s3w_bl).reshape(1, -1), lin_W,
               lin_b.reshape(1, -1))
